# baseline (speedup 1.0000x reference)
import jax
import jax.numpy as jnp
from jax import lax
from jax.experimental import pallas as pl
from jax.experimental.pallas import tpu as pltpu

N = 10000
E = 160000
B = 256
DIN = 15
DE = 4
HID = 30
EDIM = 1024
STEPS = 3
S2S_STEPS = 3
SLOPE = 0.22916667

YW = 256
ECHUNK = 2048
EP = ((E + ECHUNK - 1) // ECHUNK) * ECHUNK

_NEG = -3.4e38


def _rrelu(v):
    return jnp.where(v >= 0, v, v * SLOPE)


def _tc_head_body(x_ref, W0_ref, b0_ref, Wcat_ref, xm_ref, Y_ref):
    xm = _rrelu(jnp.dot(x_ref[...], W0_ref[...],
                        preferred_element_type=jnp.float32) + b0_ref[...])
    xm_ref[...] = xm
    Y_ref[...] = jnp.dot(xm, Wcat_ref[...], preferred_element_type=jnp.float32)


def _tc_lin0(x, W0, b0_2d, Wcat):
    return pl.pallas_call(
        _tc_head_body,
        out_shape=[jax.ShapeDtypeStruct((N, HID), jnp.float32),
                   jax.ShapeDtypeStruct((N, YW), jnp.float32)],
    )(x, W0, b0_2d, Wcat)


def _agg_body(Y_ref, src_ref, dst_ref, attr_ref, agg_ref):
    @pl.when(pl.program_id(0) == 0)
    def _init():
        agg_ref[...] = jnp.zeros_like(agg_ref)

    lane = lax.broadcasted_iota(jnp.int32, (1, 32), 1)
    unit30 = jnp.where(lane == HID, 1.0, 0.0)

    base = pl.program_id(0) * ECHUNK

    def edge_body(e, carry):
        s = src_ref[e]
        d = dst_ref[e]
        valid = jnp.where(base + e < E, 1.0, 0.0)
        row = Y_ref[pl.ds(s, 1), :]
        m = row[:, 4 * 32:5 * 32]
        for k in range(DE):
            m = m + attr_ref[DE * e + k] * row[:, k * 32:(k + 1) * 32]
        m = (m + unit30) * valid
        agg_ref[pl.ds(d, 1), :] = agg_ref[pl.ds(d, 1), :] + m
        return carry

    lax.fori_loop(0, ECHUNK, edge_body, 0)


def _tc_aggregate(Y, src, dst, attr):
    return pl.pallas_call(
        _agg_body,
        grid=(EP // ECHUNK,),
        in_specs=[
            pl.BlockSpec((N, YW), lambda i: (0, 0)),
            pl.BlockSpec((ECHUNK,), lambda i: (i,), memory_space=pltpu.SMEM),
            pl.BlockSpec((ECHUNK,), lambda i: (i,), memory_space=pltpu.SMEM),
            pl.BlockSpec((ECHUNK * DE,), lambda i: (i,),
                         memory_space=pltpu.SMEM),
        ],
        out_specs=pl.BlockSpec((N, 32), lambda i: (0, 0)),
        out_shape=jax.ShapeDtypeStruct((N, 32), jnp.float32),
    )(Y, src, dst, attr)


NB = 2000


def _tc_step_body(xm_ref, h_ref, agg_ref, Wroot_ref, bconv_ref,
                  gWihT_ref, gWhhT_ref, gbih_ref, gbhh_ref, Wcat_ref,
                  xm2_ref, h2_ref, Y2_ref):
    xm = xm_ref[...]
    h = h_ref[...]
    s = agg_ref[...]
    msum = s[:, 0:HID]
    cnt = s[:, HID:HID + 1]
    agg = msum / jnp.maximum(cnt, 1.0)
    xc = _rrelu(jnp.dot(xm, Wroot_ref[...],
                        preferred_element_type=jnp.float32)
                + agg + bconv_ref[...])
    gi = jnp.dot(xc, gWihT_ref[...],
                 preferred_element_type=jnp.float32) + gbih_ref[...]
    gh = jnp.dot(h, gWhhT_ref[...],
                 preferred_element_type=jnp.float32) + gbhh_ref[...]
    r = jax.nn.sigmoid(gi[:, 0:HID] + gh[:, 0:HID])
    z = jax.nn.sigmoid(gi[:, HID:2 * HID] + gh[:, HID:2 * HID])
    n = jnp.tanh(gi[:, 2 * HID:3 * HID] + r * gh[:, 2 * HID:3 * HID])
    h2 = (1.0 - z) * n + z * h
    xm2 = h2 + xm
    xm2_ref[...] = xm2
    h2_ref[...] = h2
    Y2_ref[...] = jnp.dot(xm2, Wcat_ref[...], preferred_element_type=jnp.float32)


def _tc_step(xm, h, agg, Wroot, bconv_2d, gWihT, gWhhT, gbih_2d, gbhh_2d, Wcat):
    full = lambda shape: pl.BlockSpec(shape, lambda i: (0,) * len(shape))
    return pl.pallas_call(
        _tc_step_body,
        grid=(N // NB,),
        in_specs=[
            pl.BlockSpec((NB, HID), lambda i: (i, 0)),
            pl.BlockSpec((NB, HID), lambda i: (i, 0)),
            pl.BlockSpec((NB, 32), lambda i: (i, 0)),
            full((HID, HID)), full((1, HID)),
            full((HID, 3 * HID)), full((HID, 3 * HID)),
            full((1, 3 * HID)), full((1, 3 * HID)),
            full((HID, YW)),
        ],
        out_specs=[
            pl.BlockSpec((NB, HID), lambda i: (i, 0)),
            pl.BlockSpec((NB, HID), lambda i: (i, 0)),
            pl.BlockSpec((NB, YW), lambda i: (i, 0)),
        ],
        out_shape=[jax.ShapeDtypeStruct((N, HID), jnp.float32),
                   jax.ShapeDtypeStruct((N, HID), jnp.float32),
                   jax.ShapeDtypeStruct((N, YW), jnp.float32)],
    )(xm, h, agg, Wroot, bconv_2d, gWihT, gWhhT, gbih_2d, gbhh_2d, Wcat)


def _tc_pool_body(xm_ref, batch_ref, lWihT_ref, lWhhT_ref, lbih_ref, lbhh_ref,
                  Wflat_ref, bflat_ref, Wout_ref, bout_ref, out_ref):
    xm = xm_ref[...]
    batch = batch_ref[...]
    iota_b = lax.broadcasted_iota(jnp.int32, (1, B), 1)
    Mb = batch == iota_b
    Mf = Mb.astype(jnp.float32)

    add = jnp.einsum('nb,nh->bh', Mf, xm,
                     preferred_element_type=jnp.float32)
    cnt = jnp.reshape(jnp.sum(Mf, axis=0), (B, 1))
    mean = add / jnp.maximum(cnt, 1.0)

    mx_cols = []
    for o in range(HID):
        col = xm[:, o:o + 1]
        masked = jnp.where(Mb, col, _NEG)
        mx_cols.append(jnp.reshape(jnp.max(masked, axis=0), (B, 1)))
    mx = jnp.concatenate(mx_cols, axis=1)

    h = jnp.zeros((B, HID), jnp.float32)
    c = jnp.zeros((B, HID), jnp.float32)
    q = jnp.zeros((B, 2 * HID), jnp.float32)
    for _ in range(S2S_STEPS):
        g = (jnp.dot(q, lWihT_ref[...], preferred_element_type=jnp.float32)
             + lbih_ref[...]
             + jnp.dot(h, lWhhT_ref[...], preferred_element_type=jnp.float32)
             + lbhh_ref[...])
        ii = jax.nn.sigmoid(g[:, 0:HID])
        ff = jax.nn.sigmoid(g[:, HID:2 * HID])
        gg = jnp.tanh(g[:, 2 * HID:3 * HID])
        oo = jax.nn.sigmoid(g[:, 3 * HID:4 * HID])
        c = ff * c + ii * gg
        h = oo * jnp.tanh(c)
        hg = jnp.einsum('nb,bh->nh', Mf, h,
                        preferred_element_type=jnp.float32)
        e = jnp.sum(xm * hg, axis=1, keepdims=True)
        masked = jnp.where(Mb, e, _NEG)
        mseg = jnp.reshape(jnp.max(masked, axis=0), (1, B))
        mg = jnp.sum(Mf * mseg, axis=1, keepdims=True)
        ex = jnp.exp(e - mg)
        sseg = jnp.reshape(jnp.sum(Mf * ex, axis=0), (1, B))
        sg = jnp.sum(Mf * sseg, axis=1, keepdims=True)
        a = ex / (sg + 1e-16)
        r = jnp.einsum('nb,nh->bh', Mf * a, xm,
                       preferred_element_type=jnp.float32)
        q = jnp.concatenate([h, r], axis=1)

    outm = jnp.concatenate([q, mean, mx, add], axis=1)
    outm = _rrelu(jnp.dot(outm, Wflat_ref[...],
                          preferred_element_type=jnp.float32) + bflat_ref[...])
    out_ref[...] = (jnp.dot(outm, Wout_ref[...],
                            preferred_element_type=jnp.float32) + bout_ref[...])


def _tc_pool(xm, batch_2d, lWihT, lWhhT, lbih_2d, lbhh_2d,
             Wflat, bflat_2d, Wout, bout_2d):
    return pl.pallas_call(
        _tc_pool_body,
        out_shape=jax.ShapeDtypeStruct((B, 1), jnp.float32),
    )(xm, batch_2d, lWihT, lWhhT, lbih_2d, lbhh_2d,
      Wflat, bflat_2d, Wout, bout_2d)


def kernel(x, edge_index, edge_attr, batch, W0, b0, We, be, Wroot, bconv,
           gWih, gWhh, gbih, gbhh, lWih, lWhh, lbih, lbhh, Wflat, bflat,
           Wout, bout):
    f32 = jnp.float32

    WeB = jnp.reshape(We, (DE, HID, HID))
    beB = jnp.reshape(be, (HID, HID))
    blocks = []
    for d in range(DE):
        blocks.append(jnp.pad(WeB[d], ((0, 0), (0, 2))))
    blocks.append(jnp.pad(beB, ((0, 0), (0, 2))))
    Wcat = jnp.pad(jnp.concatenate(blocks, axis=1),
                   ((0, 0), (0, YW - 160)))

    npad_e = EP - E
    src = jnp.concatenate([edge_index[0], jnp.zeros((npad_e,), jnp.int32)])
    dst = jnp.concatenate([edge_index[1], jnp.zeros((npad_e,), jnp.int32)])
    attr = jnp.reshape(
        jnp.concatenate([edge_attr, jnp.zeros((npad_e, DE), f32)], axis=0),
        (EP * DE,))

    b0_2d = jnp.reshape(b0, (1, HID))
    bconv_2d = jnp.reshape(bconv, (1, HID))
    gWihT = gWih.T
    gWhhT = gWhh.T
    gbih_2d = jnp.reshape(gbih, (1, 3 * HID))
    gbhh_2d = jnp.reshape(gbhh, (1, 3 * HID))
    lWihT = lWih.T
    lWhhT = lWhh.T
    lbih_2d = jnp.reshape(lbih, (1, 4 * HID))
    lbhh_2d = jnp.reshape(lbhh, (1, 4 * HID))
    bflat_2d = jnp.reshape(bflat, (1, EDIM))
    bout_2d = jnp.reshape(bout, (1, 1))
    batch_2d = jnp.reshape(batch, (N, 1))

    xm, Y = _tc_lin0(x, W0, b0_2d, Wcat)
    h = xm
    for _ in range(STEPS):
        agg = _tc_aggregate(Y, src, dst, attr)
        xm, h, Y = _tc_step(xm, h, agg, Wroot, bconv_2d,
                            gWihT, gWhhT, gbih_2d, gbhh_2d, Wcat)
    out = _tc_pool(xm, batch_2d, lWihT, lWhhT, lbih_2d, lbhh_2d,
                   Wflat, bflat_2d, Wout, bout_2d)
    return out, xm
